# baseline (speedup 1.0000x reference)
import functools

import jax
import jax.numpy as jnp
from jax import lax
from jax.experimental import pallas as pl
from jax.experimental.pallas import tpu as pltpu
from jax.experimental.pallas import tpu_sc as plsc

C, P, G, S = 8, 16, 2000, 10000
G_SEL, S_SEL = 512, 2048

NC, NS = 2, 16
NW = NC * NS
L = 16

ROWS_PER_W = C * G_SEL // NW
GENE_BLOCKS = G_SEL // ROWS_PER_W
RB = 8
NBATCH = ROWS_PER_W // RB
W_ROWS_PER_W = C * P // NW
CHUNKS = S_SEL // L
UNROLL = 8


def _gather_row(spots_v, src_ref, dst_ref, r):
    rsplat = jnp.full((L,), r, jnp.int32)

    def body(j, _):
        base = j * (L * UNROLL)
        for u in range(UNROLL):
            off = base + u * L
            idx = spots_v[pl.ds(off, L)]
            dst_ref[r, pl.ds(off, L)] = plsc.load_gather(src_ref, [rsplat, idx])
        return 0
    lax.fori_loop(0, CHUNKS // UNROLL, body, 0)


def _sc_body(lr_hbm, w_hbm, v_hbm, spots_hbm, genes_hbm,
             lr_out, w_out, v_out,
             spots_v, g_v, rows_v, gath_v, vrows_v, sem):
    cid = lax.axis_index("c")
    sid = lax.axis_index("s")
    wid = sid * NC + cid
    c = wid // GENE_BLOCKS
    gb = wid % GENE_BLOCKS
    g0 = gb * ROWS_PER_W

    pltpu.sync_copy(spots_hbm, spots_v)
    pltpu.sync_copy(genes_hbm.at[pl.ds(g0, ROWS_PER_W)], g_v)

    pltpu.async_copy(v_hbm.at[c].at[g_v], vrows_v, sem).wait()
    pltpu.sync_copy(vrows_v, v_out.at[c, pl.ds(g0, ROWS_PER_W)])

    cw = wid // (P // W_ROWS_PER_W)
    p0 = (wid % (P // W_ROWS_PER_W)) * W_ROWS_PER_W
    pltpu.sync_copy(w_hbm.at[cw, pl.ds(p0, W_ROWS_PER_W)],
                    rows_v.at[pl.ds(0, W_ROWS_PER_W)])
    for r in range(W_ROWS_PER_W):
        _gather_row(spots_v, rows_v, gath_v, r)
    pltpu.sync_copy(gath_v.at[pl.ds(0, W_ROWS_PER_W)],
                    w_out.at[cw, pl.ds(p0, W_ROWS_PER_W)])

    def batch(t, _):
        pltpu.async_copy(lr_hbm.at[c].at[g_v.at[pl.ds(t * RB, RB)]],
                         rows_v, sem).wait()
        for r in range(RB):
            _gather_row(spots_v, rows_v, gath_v, r)
        pltpu.sync_copy(gath_v, lr_out.at[c, pl.ds(g0 + t * RB, RB)])
        return 0
    lax.fori_loop(0, NBATCH, batch, 0)


_sc_gather = functools.partial(
    pl.kernel,
    mesh=plsc.VectorSubcoreMesh(core_axis_name="c", subcore_axis_name="s"),
    compiler_params=pltpu.CompilerParams(
        needs_layout_passes=False, use_tc_tiling_on_sc=False
    ),
    out_type=[
        jax.ShapeDtypeStruct((C, G_SEL, S_SEL), jnp.float32),
        jax.ShapeDtypeStruct((C, P, S_SEL), jnp.float32),
        jax.ShapeDtypeStruct((C, G_SEL, P), jnp.float32),
    ],
    scratch_types=[
        pltpu.VMEM((S_SEL,), jnp.int32),
        pltpu.VMEM((ROWS_PER_W,), jnp.int32),
        pltpu.VMEM((RB, S), jnp.float32),
        pltpu.VMEM((RB, S_SEL), jnp.float32),
        pltpu.VMEM((ROWS_PER_W, P), jnp.float32),
        pltpu.SemaphoreType.DMA,
    ],
)(_sc_body)


BG = 64


def _tc_body(lr_ref, w_ref, v_ref, o_ref):
    acc = jnp.zeros((BG, S_SEL), jnp.float32)
    for c in range(C):
        prog = jnp.dot(v_ref[c], w_ref[c], preferred_element_type=jnp.float32)
        acc = acc + jnp.exp(lr_ref[c] + prog)
    o_ref[...] = acc


_tc_combine = pl.pallas_call(
    _tc_body,
    grid=(G_SEL // BG,),
    in_specs=[
        pl.BlockSpec((C, BG, S_SEL), lambda i: (0, i, 0)),
        pl.BlockSpec((C, P, S_SEL), lambda i: (0, 0, 0)),
        pl.BlockSpec((C, BG, P), lambda i: (0, i, 0)),
    ],
    out_specs=pl.BlockSpec((BG, S_SEL), lambda i: (i, 0)),
    out_shape=jax.ShapeDtypeStruct((G_SEL, S_SEL), jnp.float32),
)


def kernel(log_rates, W, V, spots, genes):
    spots32 = spots.astype(jnp.int32)
    genes32 = genes.astype(jnp.int32)
    lr_sel, w_sel, v_sel = _sc_gather(log_rates, W, V, spots32, genes32)
    return _tc_combine(lr_sel, w_sel, v_sel)


# speedup vs baseline: 2.5350x; 2.5350x over previous
import functools

import jax
import jax.numpy as jnp
from jax import lax
from jax.experimental import pallas as pl
from jax.experimental.pallas import tpu as pltpu
from jax.experimental.pallas import tpu_sc as plsc

C, P, G, S = 8, 16, 2000, 10000
G_SEL, S_SEL = 512, 2048

NC, NS = 2, 16
NW = NC * NS
L = 16

ROWS_PER_W = C * G_SEL // NW
GENE_BLOCKS = G_SEL // ROWS_PER_W
RB = 8
NBATCH = ROWS_PER_W // RB
W_ROWS_PER_W = C * P // NW
CHUNKS = S_SEL // L
UNROLL = 8

_LANE_IOTA = None


def _extract(vec, lane):
    sel = jnp.where(lax.iota(jnp.int32, L) == lane, vec, -1)
    return jnp.max(sel)


def _gather_row(spots_v, src_ref, dst_ref, r):
    rsplat = jnp.full((L,), r, jnp.int32)

    def body(j, _):
        base = j * (L * UNROLL)
        for u in range(UNROLL):
            off = base + u * L
            idx = spots_v[pl.ds(off, L)]
            dst_ref[r, pl.ds(off, L)] = plsc.load_gather(src_ref, [rsplat, idx])
        return 0
    lax.fori_loop(0, CHUNKS // UNROLL, body, 0)


def _sc_body(lr_hbm, w_hbm, v_hbm, spots_hbm, genes_hbm,
             lr_out, w_out, v_out,
             spots_v, g_v, rows_v, gath_v, vrows_v, sem):
    cid = lax.axis_index("c")
    sid = lax.axis_index("s")
    wid = sid * NC + cid
    c = wid // GENE_BLOCKS
    gb = wid % GENE_BLOCKS
    g0 = gb * ROWS_PER_W

    pltpu.sync_copy(spots_hbm, spots_v)
    pltpu.sync_copy(genes_hbm.at[pl.ds(g0, ROWS_PER_W)], g_v)

    vcopies = []
    for q in range(ROWS_PER_W // L):
        vec = g_v[pl.ds(q * L, L)]
        for u in range(L):
            g = _extract(vec, u)
            vcopies.append(
                pltpu.async_copy(v_hbm.at[c, g], vrows_v.at[q * L + u], sem))
    for cp in vcopies:
        cp.wait()
    pltpu.sync_copy(vrows_v, v_out.at[c, pl.ds(g0, ROWS_PER_W)])

    cw = wid // (P // W_ROWS_PER_W)
    p0 = (wid % (P // W_ROWS_PER_W)) * W_ROWS_PER_W
    wcopies = [
        pltpu.async_copy(w_hbm.at[cw, p0 + r], rows_v.at[r], sem)
        for r in range(W_ROWS_PER_W)
    ]
    for cp in wcopies:
        cp.wait()
    for r in range(W_ROWS_PER_W):
        _gather_row(spots_v, rows_v, gath_v, r)
    pltpu.sync_copy(gath_v.at[pl.ds(0, W_ROWS_PER_W)],
                    w_out.at[cw, pl.ds(p0, W_ROWS_PER_W)])

    def batch(t, _):
        vec = g_v[pl.ds((t // 2) * L, L)]
        lane0 = (t % 2) * RB
        for r in range(RB):
            g = _extract(vec, lane0 + r)
            pltpu.async_copy(lr_hbm.at[c, g], rows_v.at[r], sem)
        for r in range(RB):
            pltpu.make_async_copy(
                lr_hbm.at[c, 0], rows_v.at[r], sem).wait()
        for r in range(RB):
            _gather_row(spots_v, rows_v, gath_v, r)
        pltpu.sync_copy(gath_v, lr_out.at[c, pl.ds(g0 + t * RB, RB)])
        return 0
    lax.fori_loop(0, NBATCH, batch, 0)


_sc_gather = functools.partial(
    pl.kernel,
    mesh=plsc.VectorSubcoreMesh(core_axis_name="c", subcore_axis_name="s"),
    compiler_params=pltpu.CompilerParams(needs_layout_passes=False),
    out_type=[
        jax.ShapeDtypeStruct((C, G_SEL, S_SEL), jnp.float32),
        jax.ShapeDtypeStruct((C, P, S_SEL), jnp.float32),
        jax.ShapeDtypeStruct((C, G_SEL, P), jnp.float32),
    ],
    scratch_types=[
        pltpu.VMEM((S_SEL,), jnp.int32),
        pltpu.VMEM((ROWS_PER_W,), jnp.int32),
        pltpu.VMEM((RB, S), jnp.float32),
        pltpu.VMEM((RB, S_SEL), jnp.float32),
        pltpu.VMEM((ROWS_PER_W, P), jnp.float32),
        pltpu.SemaphoreType.DMA,
    ],
)(_sc_body)


BG = 64


def _tc_body(lr_ref, w_ref, v_ref, o_ref):
    acc = jnp.zeros((BG, S_SEL), jnp.float32)
    for c in range(C):
        prog = jnp.dot(v_ref[c], w_ref[c], preferred_element_type=jnp.float32)
        acc = acc + jnp.exp(lr_ref[c] + prog)
    o_ref[...] = acc


_tc_combine = pl.pallas_call(
    _tc_body,
    grid=(G_SEL // BG,),
    in_specs=[
        pl.BlockSpec((C, BG, S_SEL), lambda i: (0, i, 0)),
        pl.BlockSpec((C, P, S_SEL), lambda i: (0, 0, 0)),
        pl.BlockSpec((C, BG, P), lambda i: (0, i, 0)),
    ],
    out_specs=pl.BlockSpec((BG, S_SEL), lambda i: (i, 0)),
    out_shape=jax.ShapeDtypeStruct((G_SEL, S_SEL), jnp.float32),
)


def kernel(log_rates, W, V, spots, genes):
    spots32 = spots.astype(jnp.int32)
    genes32 = genes.astype(jnp.int32)
    lr_sel, w_sel, v_sel = _sc_gather(log_rates, W, V, spots32, genes32)
    return _tc_combine(lr_sel, w_sel, v_sel)


# speedup vs baseline: 2.6359x; 1.0398x over previous
import functools

import jax
import jax.numpy as jnp
from jax import lax
from jax.experimental import pallas as pl
from jax.experimental.pallas import tpu as pltpu
from jax.experimental.pallas import tpu_sc as plsc

C, P, G, S = 8, 16, 2000, 10000
G_SEL, S_SEL = 512, 2048

NC, NS = 2, 16
NW = NC * NS
L = 16

ROWS_PER_W = C * G_SEL // NW
GENE_BLOCKS = G_SEL // ROWS_PER_W
RB = 8
HB = 4
NHB = ROWS_PER_W // HB
HPW = L // HB
W_ROWS_PER_W = C * P // NW
CHUNKS = S_SEL // L
UNROLL = 8

_LANE_IOTA = None


def _extract(vec, lane):
    sel = jnp.where(lax.iota(jnp.int32, L) == lane, vec, -1)
    return jnp.max(sel)


def _gather_row(spots_v, src_ref, dst_ref, r):
    rsplat = jnp.full((L,), r, jnp.int32)

    def body(j, _):
        base = j * (L * UNROLL)
        for u in range(UNROLL):
            off = base + u * L
            idx = spots_v[pl.ds(off, L)]
            dst_ref[r, pl.ds(off, L)] = plsc.load_gather(src_ref, [rsplat, idx])
        return 0
    lax.fori_loop(0, CHUNKS // UNROLL, body, 0)


def _sc_body(lr_hbm, w_hbm, v_hbm, spots_hbm, genes_hbm,
             lr_out, w_out, v_out,
             spots_v, g_v, rows_v, gath_v, vrows_v, sem):
    cid = lax.axis_index("c")
    sid = lax.axis_index("s")
    wid = sid * NC + cid
    c = wid // GENE_BLOCKS
    gb = wid % GENE_BLOCKS
    g0 = gb * ROWS_PER_W

    pltpu.sync_copy(spots_hbm, spots_v)
    pltpu.sync_copy(genes_hbm.at[pl.ds(g0, ROWS_PER_W)], g_v)

    vcopies = []
    for q in range(ROWS_PER_W // L):
        vec = g_v[pl.ds(q * L, L)]
        for u in range(L):
            g = _extract(vec, u)
            vcopies.append(
                pltpu.async_copy(v_hbm.at[c, g], vrows_v.at[q * L + u], sem))
    for cp in vcopies:
        cp.wait()
    pltpu.sync_copy(vrows_v, v_out.at[c, pl.ds(g0, ROWS_PER_W)])

    cw = wid // (P // W_ROWS_PER_W)
    p0 = (wid % (P // W_ROWS_PER_W)) * W_ROWS_PER_W
    wcopies = [
        pltpu.async_copy(w_hbm.at[cw, p0 + r], rows_v.at[r], sem)
        for r in range(W_ROWS_PER_W)
    ]
    for cp in wcopies:
        cp.wait()
    for r in range(W_ROWS_PER_W):
        _gather_row(spots_v, rows_v, gath_v, r)
    pltpu.sync_copy(gath_v.at[pl.ds(0, W_ROWS_PER_W)],
                    w_out.at[cw, pl.ds(p0, W_ROWS_PER_W)])

    def fire(t, half):
        vec = g_v[pl.ds((t // HPW) * L, L)]
        lane0 = (t % HPW) * HB
        for i in range(HB):
            g = _extract(vec, lane0 + i)
            pltpu.async_copy(lr_hbm.at[c, g], rows_v.at[half * HB + i], sem)

    fire(0, 0)

    def batch(t, _):
        half = t % 2
        for i in range(HB):
            pltpu.make_async_copy(
                lr_hbm.at[c, 0], rows_v.at[half * HB + i], sem).wait()

        @pl.when(t + 1 < NHB)
        def _():
            fire(t + 1, 1 - half)

        for i in range(HB):
            _gather_row(spots_v, rows_v, gath_v, half * HB + i)

        @pl.when(half == 1)
        def _():
            off = pl.multiple_of(g0 + (t - 1) * HB, 2 * HB)
            pltpu.sync_copy(gath_v, lr_out.at[c, pl.ds(off, 2 * HB)])
        return 0
    lax.fori_loop(0, NHB, batch, 0)


_sc_gather = functools.partial(
    pl.kernel,
    mesh=plsc.VectorSubcoreMesh(core_axis_name="c", subcore_axis_name="s"),
    compiler_params=pltpu.CompilerParams(needs_layout_passes=False),
    out_type=[
        jax.ShapeDtypeStruct((C, G_SEL, S_SEL), jnp.float32),
        jax.ShapeDtypeStruct((C, P, S_SEL), jnp.float32),
        jax.ShapeDtypeStruct((C, G_SEL, P), jnp.float32),
    ],
    scratch_types=[
        pltpu.VMEM((S_SEL,), jnp.int32),
        pltpu.VMEM((ROWS_PER_W,), jnp.int32),
        pltpu.VMEM((RB, S), jnp.float32),
        pltpu.VMEM((RB, S_SEL), jnp.float32),
        pltpu.VMEM((ROWS_PER_W, P), jnp.float32),
        pltpu.SemaphoreType.DMA,
    ],
)(_sc_body)


BG = 64


def _tc_body(lr_ref, w_ref, v_ref, o_ref):
    acc = jnp.zeros((BG, S_SEL), jnp.float32)
    for c in range(C):
        prog = jnp.dot(v_ref[c], w_ref[c], preferred_element_type=jnp.float32)
        acc = acc + jnp.exp(lr_ref[c] + prog)
    o_ref[...] = acc


_tc_combine = pl.pallas_call(
    _tc_body,
    grid=(G_SEL // BG,),
    in_specs=[
        pl.BlockSpec((C, BG, S_SEL), lambda i: (0, i, 0)),
        pl.BlockSpec((C, P, S_SEL), lambda i: (0, 0, 0)),
        pl.BlockSpec((C, BG, P), lambda i: (0, i, 0)),
    ],
    out_specs=pl.BlockSpec((BG, S_SEL), lambda i: (i, 0)),
    out_shape=jax.ShapeDtypeStruct((G_SEL, S_SEL), jnp.float32),
)


def kernel(log_rates, W, V, spots, genes):
    spots32 = spots.astype(jnp.int32)
    genes32 = genes.astype(jnp.int32)
    lr_sel, w_sel, v_sel = _sc_gather(log_rates, W, V, spots32, genes32)
    return _tc_combine(lr_sel, w_sel, v_sel)


# speedup vs baseline: 2.6454x; 1.0036x over previous
import functools

import jax
import jax.numpy as jnp
from jax import lax
from jax.experimental import pallas as pl
from jax.experimental.pallas import tpu as pltpu
from jax.experimental.pallas import tpu_sc as plsc

C, P, G, S = 8, 16, 2000, 10000
G_SEL, S_SEL = 512, 2048

NC, NS = 2, 16
NW = NC * NS
L = 16

ROWS_PER_W = C * G_SEL // NW
GENE_BLOCKS = G_SEL // ROWS_PER_W
RB = 8
HB = 4
NHB = ROWS_PER_W // HB
HPW = L // HB
W_ROWS_PER_W = C * P // NW
CHUNKS = S_SEL // L
UNROLL = 8

_LANE_IOTA = None


def _extract(vec, lane):
    sel = jnp.where(lax.iota(jnp.int32, L) == lane, vec, -1)
    return jnp.max(sel)


def _gather_row(spots_v, src_ref, dst_ref, r):
    rsplat = jnp.full((L,), r, jnp.int32)

    def body(j, _):
        base = j * (L * UNROLL)
        for u in range(UNROLL):
            off = base + u * L
            idx = spots_v[pl.ds(off, L)]
            dst_ref[r, pl.ds(off, L)] = plsc.load_gather(src_ref, [rsplat, idx])
        return 0
    lax.fori_loop(0, CHUNKS // UNROLL, body, 0)


def _sc_body(lr_hbm, w_hbm, v_hbm, spots_hbm, genes_hbm,
             lr_out, w_out, v_out,
             spots_v, g_v, rows_v, gath_v, vrows_v, sem, sem2):
    cid = lax.axis_index("c")
    sid = lax.axis_index("s")
    wid = sid * NC + cid
    c = wid // GENE_BLOCKS
    gb = wid % GENE_BLOCKS
    g0 = gb * ROWS_PER_W

    pltpu.sync_copy(spots_hbm, spots_v)
    pltpu.sync_copy(genes_hbm.at[pl.ds(g0, ROWS_PER_W)], g_v)

    vcopies = []
    for q in range(ROWS_PER_W // L):
        vec = g_v[pl.ds(q * L, L)]
        for u in range(L):
            g = _extract(vec, u)
            vcopies.append(
                pltpu.async_copy(v_hbm.at[c, g], vrows_v.at[q * L + u], sem))
    for cp in vcopies:
        cp.wait()
    pltpu.sync_copy(vrows_v, v_out.at[c, pl.ds(g0, ROWS_PER_W)])

    cw = wid // (P // W_ROWS_PER_W)
    p0 = (wid % (P // W_ROWS_PER_W)) * W_ROWS_PER_W
    wcopies = [
        pltpu.async_copy(w_hbm.at[cw, p0 + r], rows_v.at[r], sem)
        for r in range(W_ROWS_PER_W)
    ]
    for cp in wcopies:
        cp.wait()
    for r in range(W_ROWS_PER_W):
        _gather_row(spots_v, rows_v, gath_v, r)
    pltpu.sync_copy(gath_v.at[pl.ds(0, W_ROWS_PER_W)],
                    w_out.at[cw, pl.ds(p0, W_ROWS_PER_W)])

    def fire(t, half, fsem):
        vec = g_v[pl.ds((t // HPW) * L, L)]
        lane0 = (t % HPW) * HB
        for i in range(HB):
            g = _extract(vec, lane0 + i)
            pltpu.async_copy(lr_hbm.at[c, g], rows_v.at[half * HB + i], fsem)

    def process(t, half, my_sem, other_sem):
        @pl.when(t + 1 < NHB)
        def _():
            fire(t + 1, 1 - half, other_sem)

        for i in range(HB):
            pltpu.make_async_copy(
                lr_hbm.at[c, 0], rows_v.at[half * HB + i], my_sem).wait()
        for i in range(HB):
            _gather_row(spots_v, rows_v, gath_v, half * HB + i)

        if half == 1:
            off = pl.multiple_of(g0 + (t - 1) * HB, 2 * HB)
            pltpu.sync_copy(gath_v, lr_out.at[c, pl.ds(off, 2 * HB)])

    fire(0, 0, sem)

    def batch(t, _):
        @pl.when(t % 2 == 0)
        def _():
            process(t, 0, sem, sem2)

        @pl.when(t % 2 == 1)
        def _():
            process(t, 1, sem2, sem)
        return 0
    lax.fori_loop(0, NHB, batch, 0)


_sc_gather = functools.partial(
    pl.kernel,
    mesh=plsc.VectorSubcoreMesh(core_axis_name="c", subcore_axis_name="s"),
    compiler_params=pltpu.CompilerParams(needs_layout_passes=False),
    out_type=[
        jax.ShapeDtypeStruct((C, G_SEL, S_SEL), jnp.float32),
        jax.ShapeDtypeStruct((C, P, S_SEL), jnp.float32),
        jax.ShapeDtypeStruct((C, G_SEL, P), jnp.float32),
    ],
    scratch_types=[
        pltpu.VMEM((S_SEL,), jnp.int32),
        pltpu.VMEM((ROWS_PER_W,), jnp.int32),
        pltpu.VMEM((RB, S), jnp.float32),
        pltpu.VMEM((RB, S_SEL), jnp.float32),
        pltpu.VMEM((ROWS_PER_W, P), jnp.float32),
        pltpu.SemaphoreType.DMA,
        pltpu.SemaphoreType.DMA,
    ],
)(_sc_body)


BG = 64


def _tc_body(lr_ref, w_ref, v_ref, o_ref):
    acc = jnp.zeros((BG, S_SEL), jnp.float32)
    for c in range(C):
        prog = jnp.dot(v_ref[c], w_ref[c], preferred_element_type=jnp.float32)
        acc = acc + jnp.exp(lr_ref[c] + prog)
    o_ref[...] = acc


_tc_combine = pl.pallas_call(
    _tc_body,
    grid=(G_SEL // BG,),
    in_specs=[
        pl.BlockSpec((C, BG, S_SEL), lambda i: (0, i, 0)),
        pl.BlockSpec((C, P, S_SEL), lambda i: (0, 0, 0)),
        pl.BlockSpec((C, BG, P), lambda i: (0, i, 0)),
    ],
    out_specs=pl.BlockSpec((BG, S_SEL), lambda i: (i, 0)),
    out_shape=jax.ShapeDtypeStruct((G_SEL, S_SEL), jnp.float32),
)


def kernel(log_rates, W, V, spots, genes):
    spots32 = spots.astype(jnp.int32)
    genes32 = genes.astype(jnp.int32)
    lr_sel, w_sel, v_sel = _sc_gather(log_rates, W, V, spots32, genes32)
    return _tc_combine(lr_sel, w_sel, v_sel)


# speedup vs baseline: 2.6519x; 1.0024x over previous
import functools

import jax
import jax.numpy as jnp
from jax import lax
from jax.experimental import pallas as pl
from jax.experimental.pallas import tpu as pltpu
from jax.experimental.pallas import tpu_sc as plsc

C, P, G, S = 8, 16, 2000, 10000
G_SEL, S_SEL = 512, 2048

NC, NS = 2, 16
NW = NC * NS
L = 16

ROWS_PER_W = C * G_SEL // NW
GENE_BLOCKS = G_SEL // ROWS_PER_W
RB = 8
HB = 4
NHB = ROWS_PER_W // HB
HPW = L // HB
S_HALF = 4992
W_ROWS_PER_W = C * P // NW
CHUNKS = S_SEL // L
UNROLL = 8

_LANE_IOTA = None


def _extract(vec, lane):
    sel = jnp.where(lax.iota(jnp.int32, L) == lane, vec, -1)
    return jnp.max(sel)


def _gather_row(spots_v, src_ref, dst_ref, r):
    rsplat = jnp.full((L,), r, jnp.int32)

    def body(j, _):
        base = j * (L * UNROLL)
        for u in range(UNROLL):
            off = base + u * L
            idx = spots_v[pl.ds(off, L)]
            dst_ref[r, pl.ds(off, L)] = plsc.load_gather(src_ref, [rsplat, idx])
        return 0
    lax.fori_loop(0, CHUNKS // UNROLL, body, 0)


def _sc_body(lr_hbm, w_hbm, v_hbm, spots_hbm, genes_hbm,
             lr_out, w_out, v_out,
             spots_v, g_v, rows_v, gath_v, vrows_v, sem, sem2):
    cid = lax.axis_index("c")
    sid = lax.axis_index("s")
    wid = sid * NC + cid
    c = wid // GENE_BLOCKS
    gb = wid % GENE_BLOCKS
    g0 = gb * ROWS_PER_W

    pltpu.sync_copy(spots_hbm, spots_v)
    pltpu.sync_copy(genes_hbm.at[pl.ds(g0, ROWS_PER_W)], g_v)

    vcopies = []
    for q in range(ROWS_PER_W // L):
        vec = g_v[pl.ds(q * L, L)]
        for u in range(L):
            g = _extract(vec, u)
            vcopies.append(
                pltpu.async_copy(v_hbm.at[c, g], vrows_v.at[q * L + u], sem))
    for cp in vcopies:
        cp.wait()
    pltpu.sync_copy(vrows_v, v_out.at[c, pl.ds(g0, ROWS_PER_W)])

    cw = wid // (P // W_ROWS_PER_W)
    p0 = (wid % (P // W_ROWS_PER_W)) * W_ROWS_PER_W
    wcopies = [
        pltpu.async_copy(w_hbm.at[cw, p0 + r], rows_v.at[r], sem)
        for r in range(W_ROWS_PER_W)
    ]
    for cp in wcopies:
        cp.wait()
    for r in range(W_ROWS_PER_W):
        _gather_row(spots_v, rows_v, gath_v, r)
    pltpu.sync_copy(gath_v.at[pl.ds(0, W_ROWS_PER_W)],
                    w_out.at[cw, pl.ds(p0, W_ROWS_PER_W)])

    def fire(t, half, fsem):
        vec = g_v[pl.ds((t // HPW) * L, L)]
        lane0 = (t % HPW) * HB
        for i in range(HB):
            g = _extract(vec, lane0 + i)
            pltpu.async_copy(lr_hbm.at[c, g, pl.ds(0, S_HALF)],
                             rows_v.at[half * HB + i, pl.ds(0, S_HALF)], fsem)
            pltpu.async_copy(lr_hbm.at[c, g, pl.ds(S_HALF, S - S_HALF)],
                             rows_v.at[half * HB + i, pl.ds(S_HALF, S - S_HALF)],
                             fsem)

    def process(t, half, my_sem, other_sem):
        @pl.when(t + 1 < NHB)
        def _():
            fire(t + 1, 1 - half, other_sem)

        for i in range(HB):
            pltpu.make_async_copy(
                lr_hbm.at[c, 0], rows_v.at[half * HB + i], my_sem).wait()
        for i in range(HB):
            _gather_row(spots_v, rows_v, gath_v, half * HB + i)

        if half == 1:
            off = pl.multiple_of(g0 + (t - 1) * HB, 2 * HB)
            pltpu.sync_copy(gath_v, lr_out.at[c, pl.ds(off, 2 * HB)])

    fire(0, 0, sem)

    def batch(t, _):
        @pl.when(t % 2 == 0)
        def _():
            process(t, 0, sem, sem2)

        @pl.when(t % 2 == 1)
        def _():
            process(t, 1, sem2, sem)
        return 0
    lax.fori_loop(0, NHB, batch, 0)


_sc_gather = functools.partial(
    pl.kernel,
    mesh=plsc.VectorSubcoreMesh(core_axis_name="c", subcore_axis_name="s"),
    compiler_params=pltpu.CompilerParams(needs_layout_passes=False),
    out_type=[
        jax.ShapeDtypeStruct((C, G_SEL, S_SEL), jnp.float32),
        jax.ShapeDtypeStruct((C, P, S_SEL), jnp.float32),
        jax.ShapeDtypeStruct((C, G_SEL, P), jnp.float32),
    ],
    scratch_types=[
        pltpu.VMEM((S_SEL,), jnp.int32),
        pltpu.VMEM((ROWS_PER_W,), jnp.int32),
        pltpu.VMEM((RB, S), jnp.float32),
        pltpu.VMEM((RB, S_SEL), jnp.float32),
        pltpu.VMEM((ROWS_PER_W, P), jnp.float32),
        pltpu.SemaphoreType.DMA,
        pltpu.SemaphoreType.DMA,
    ],
)(_sc_body)


BG = 64


def _tc_body(lr_ref, w_ref, v_ref, o_ref):
    acc = jnp.zeros((BG, S_SEL), jnp.float32)
    for c in range(C):
        prog = jnp.dot(v_ref[c], w_ref[c], preferred_element_type=jnp.float32)
        acc = acc + jnp.exp(lr_ref[c] + prog)
    o_ref[...] = acc


_tc_combine = pl.pallas_call(
    _tc_body,
    grid=(G_SEL // BG,),
    in_specs=[
        pl.BlockSpec((C, BG, S_SEL), lambda i: (0, i, 0)),
        pl.BlockSpec((C, P, S_SEL), lambda i: (0, 0, 0)),
        pl.BlockSpec((C, BG, P), lambda i: (0, i, 0)),
    ],
    out_specs=pl.BlockSpec((BG, S_SEL), lambda i: (i, 0)),
    out_shape=jax.ShapeDtypeStruct((G_SEL, S_SEL), jnp.float32),
)


def kernel(log_rates, W, V, spots, genes):
    spots32 = spots.astype(jnp.int32)
    genes32 = genes.astype(jnp.int32)
    lr_sel, w_sel, v_sel = _sc_gather(log_rates, W, V, spots32, genes32)
    return _tc_combine(lr_sel, w_sel, v_sel)


# speedup vs baseline: 5.9584x; 2.2469x over previous
import functools

import jax
import jax.numpy as jnp
from jax import lax
from jax.experimental import pallas as pl
from jax.experimental.pallas import tpu as pltpu
from jax.experimental.pallas import tpu_sc as plsc

C, P, G, S = 8, 16, 2000, 10000
G_SEL, S_SEL = 512, 2048

NC, NS = 2, 16
NW = NC * NS
L = 16

ROWS_PER_W = C * G_SEL // NW
GENE_BLOCKS = G_SEL // ROWS_PER_W
RB = 8
HB = 4
NHB = ROWS_PER_W // HB
HPW = L // HB
S_HALF = 4992
W_ROWS_PER_W = C * P // NW
CHUNKS = S_SEL // L
UNROLL = 8

_LANE_IOTA = None


def _extract(vec, lane):
    sel = jnp.where(lax.iota(jnp.int32, L) == lane, vec, -1)
    return jnp.max(sel)


def _gather_row(spots_v, src_ref, dst_ref, r):
    rsplat = jnp.full((L,), r, jnp.int32)

    @plsc.parallel_loop(0, CHUNKS, unroll=UNROLL)
    def _(j):
        off = j * L
        idx = spots_v[pl.ds(off, L)]
        dst_ref[r, pl.ds(off, L)] = plsc.load_gather(src_ref, [rsplat, idx])


def _sc_body(lr_hbm, w_hbm, v_hbm, spots_hbm, genes_hbm,
             lr_out, w_out, v_out,
             spots_v, g_v, rows_v, gath_v, vrows_v, sem, sem2):
    cid = lax.axis_index("c")
    sid = lax.axis_index("s")
    wid = sid * NC + cid
    c = wid // GENE_BLOCKS
    gb = wid % GENE_BLOCKS
    g0 = gb * ROWS_PER_W

    pltpu.sync_copy(spots_hbm, spots_v)
    pltpu.sync_copy(genes_hbm.at[pl.ds(g0, ROWS_PER_W)], g_v)

    vcopies = []
    for q in range(ROWS_PER_W // L):
        vec = g_v[pl.ds(q * L, L)]
        for u in range(L):
            g = _extract(vec, u)
            vcopies.append(
                pltpu.async_copy(v_hbm.at[c, g], vrows_v.at[q * L + u], sem))
    for cp in vcopies:
        cp.wait()
    pltpu.sync_copy(vrows_v, v_out.at[c, pl.ds(g0, ROWS_PER_W)])

    cw = wid // (P // W_ROWS_PER_W)
    p0 = (wid % (P // W_ROWS_PER_W)) * W_ROWS_PER_W
    wcopies = [
        pltpu.async_copy(w_hbm.at[cw, p0 + r], rows_v.at[r], sem)
        for r in range(W_ROWS_PER_W)
    ]
    for cp in wcopies:
        cp.wait()
    for r in range(W_ROWS_PER_W):
        _gather_row(spots_v, rows_v, gath_v, r)
    pltpu.sync_copy(gath_v.at[pl.ds(0, W_ROWS_PER_W)],
                    w_out.at[cw, pl.ds(p0, W_ROWS_PER_W)])

    def fire(t, half, fsem):
        vec = g_v[pl.ds((t // HPW) * L, L)]
        lane0 = (t % HPW) * HB
        for i in range(HB):
            g = _extract(vec, lane0 + i)
            pltpu.async_copy(lr_hbm.at[c, g, pl.ds(0, S_HALF)],
                             rows_v.at[half * HB + i, pl.ds(0, S_HALF)], fsem)
            pltpu.async_copy(lr_hbm.at[c, g, pl.ds(S_HALF, S - S_HALF)],
                             rows_v.at[half * HB + i, pl.ds(S_HALF, S - S_HALF)],
                             fsem)

    def process(t, half, my_sem, other_sem):
        @pl.when(t + 1 < NHB)
        def _():
            fire(t + 1, 1 - half, other_sem)

        for i in range(HB):
            pltpu.make_async_copy(
                lr_hbm.at[c, 0], rows_v.at[half * HB + i], my_sem).wait()
        for i in range(HB):
            _gather_row(spots_v, rows_v, gath_v, half * HB + i)

        if half == 1:
            off = pl.multiple_of(g0 + (t - 1) * HB, 2 * HB)
            pltpu.sync_copy(gath_v, lr_out.at[c, pl.ds(off, 2 * HB)])

    fire(0, 0, sem)

    def batch(t, _):
        @pl.when(t % 2 == 0)
        def _():
            process(t, 0, sem, sem2)

        @pl.when(t % 2 == 1)
        def _():
            process(t, 1, sem2, sem)
        return 0
    lax.fori_loop(0, NHB, batch, 0)


_sc_gather = functools.partial(
    pl.kernel,
    mesh=plsc.VectorSubcoreMesh(core_axis_name="c", subcore_axis_name="s"),
    compiler_params=pltpu.CompilerParams(needs_layout_passes=False),
    out_type=[
        jax.ShapeDtypeStruct((C, G_SEL, S_SEL), jnp.float32),
        jax.ShapeDtypeStruct((C, P, S_SEL), jnp.float32),
        jax.ShapeDtypeStruct((C, G_SEL, P), jnp.float32),
    ],
    scratch_types=[
        pltpu.VMEM((S_SEL,), jnp.int32),
        pltpu.VMEM((ROWS_PER_W,), jnp.int32),
        pltpu.VMEM((RB, S), jnp.float32),
        pltpu.VMEM((RB, S_SEL), jnp.float32),
        pltpu.VMEM((ROWS_PER_W, P), jnp.float32),
        pltpu.SemaphoreType.DMA,
        pltpu.SemaphoreType.DMA,
    ],
)(_sc_body)


BG = 64


def _tc_body(lr_ref, w_ref, v_ref, o_ref):
    acc = jnp.zeros((BG, S_SEL), jnp.float32)
    for c in range(C):
        prog = jnp.dot(v_ref[c], w_ref[c], preferred_element_type=jnp.float32)
        acc = acc + jnp.exp(lr_ref[c] + prog)
    o_ref[...] = acc


_tc_combine = pl.pallas_call(
    _tc_body,
    grid=(G_SEL // BG,),
    in_specs=[
        pl.BlockSpec((C, BG, S_SEL), lambda i: (0, i, 0)),
        pl.BlockSpec((C, P, S_SEL), lambda i: (0, 0, 0)),
        pl.BlockSpec((C, BG, P), lambda i: (0, i, 0)),
    ],
    out_specs=pl.BlockSpec((BG, S_SEL), lambda i: (i, 0)),
    out_shape=jax.ShapeDtypeStruct((G_SEL, S_SEL), jnp.float32),
)


def kernel(log_rates, W, V, spots, genes):
    spots32 = spots.astype(jnp.int32)
    genes32 = genes.astype(jnp.int32)
    lr_sel, w_sel, v_sel = _sc_gather(log_rates, W, V, spots32, genes32)
    return _tc_combine(lr_sel, w_sel, v_sel)


# speedup vs baseline: 6.3539x; 1.0664x over previous
import functools

import jax
import jax.numpy as jnp
from jax import lax
from jax.experimental import pallas as pl
from jax.experimental.pallas import tpu as pltpu
from jax.experimental.pallas import tpu_sc as plsc

C, P, G, S = 8, 16, 2000, 10000
G_SEL, S_SEL = 512, 2048

NC, NS = 2, 16
NW = NC * NS
L = 16

ROWS_PER_W = C * G_SEL // NW
GENE_BLOCKS = G_SEL // ROWS_PER_W
RB = 8
HB = 4
NHB = ROWS_PER_W // HB
HPW = L // HB
S_HALF = 4992
VR = 64
W_ROWS_PER_W = C * P // NW
CHUNKS = S_SEL // L
UNROLL = 8

_LANE_IOTA = None


def _extract(vec, lane):
    sel = jnp.where(lax.iota(jnp.int32, L) == lane, vec, -1)
    return jnp.max(sel)


def _gather_row(spots_v, src_ref, dst_ref, r, rd):
    rsplat = jnp.full((L,), r, jnp.int32)

    @plsc.parallel_loop(0, CHUNKS, unroll=UNROLL)
    def _(j):
        off = j * L
        idx = spots_v[pl.ds(off, L)]
        dst_ref[rd, pl.ds(off, L)] = plsc.load_gather(src_ref, [rsplat, idx])


def _sc_body(lr_hbm, w_hbm, v_hbm, spots_hbm, genes_hbm,
             lr_out, w_out, v_out,
             spots_v, g_v, rows_v, gath_v, vrows_v, sem, sem2, wsem):
    cid = lax.axis_index("c")
    sid = lax.axis_index("s")
    wid = sid * NC + cid
    c = wid // GENE_BLOCKS
    gb = wid % GENE_BLOCKS
    g0 = gb * ROWS_PER_W

    pltpu.sync_copy(spots_hbm, spots_v)
    pltpu.sync_copy(genes_hbm.at[pl.ds(g0, ROWS_PER_W)], g_v)

    for h in range(2):
        vcopies = []
        for q in range(VR // L):
            vec = g_v[pl.ds(h * VR + q * L, L)]
            for u in range(L):
                g = _extract(vec, u)
                vcopies.append(
                    pltpu.async_copy(v_hbm.at[c, g], vrows_v.at[q * L + u],
                                     sem))
        for cp in vcopies:
            cp.wait()
        pltpu.sync_copy(vrows_v, v_out.at[c, pl.ds(g0 + h * VR, VR)])

    cw = wid // (P // W_ROWS_PER_W)
    p0 = (wid % (P // W_ROWS_PER_W)) * W_ROWS_PER_W
    wcopies = [
        pltpu.async_copy(w_hbm.at[cw, p0 + r], rows_v.at[r], sem)
        for r in range(W_ROWS_PER_W)
    ]
    for cp in wcopies:
        cp.wait()
    for r in range(W_ROWS_PER_W):
        _gather_row(spots_v, rows_v, gath_v, r, r)
    pltpu.sync_copy(gath_v.at[pl.ds(0, W_ROWS_PER_W)],
                    w_out.at[cw, pl.ds(p0, W_ROWS_PER_W)])

    def fire(t, half, fsem):
        vec = g_v[pl.ds((t // HPW) * L, L)]
        lane0 = (t % HPW) * HB
        for i in range(HB):
            g = _extract(vec, lane0 + i)
            pltpu.async_copy(lr_hbm.at[c, g, pl.ds(0, S_HALF)],
                             rows_v.at[half * HB + i, pl.ds(0, S_HALF)], fsem)
            pltpu.async_copy(lr_hbm.at[c, g, pl.ds(S_HALF, S - S_HALF)],
                             rows_v.at[half * HB + i, pl.ds(S_HALF, S - S_HALF)],
                             fsem)

    def process(t, half, my_sem, other_sem):
        @pl.when(t + 1 < NHB)
        def _():
            fire(t + 1, 1 - half, other_sem)

        for i in range(HB):
            pltpu.make_async_copy(
                lr_hbm.at[c, 0], rows_v.at[half * HB + i], my_sem).wait()
        gbase = ((t // 2) % 2) * (2 * HB)
        for i in range(HB):
            _gather_row(spots_v, rows_v, gath_v, half * HB + i,
                        gbase + half * HB + i)

        if half == 1:
            @pl.when(t > 1)
            def _():
                pltpu.make_async_copy(
                    lr_out.at[c, pl.ds(0, 2 * HB)],
                    gath_v.at[pl.ds(0, 2 * HB)], wsem).wait()
            off = pl.multiple_of(g0 + (t - 1) * HB, 2 * HB)
            gb8 = pl.multiple_of(gbase, 2 * HB)
            pltpu.async_copy(gath_v.at[pl.ds(gb8, 2 * HB)],
                             lr_out.at[c, pl.ds(off, 2 * HB)], wsem)

    fire(0, 0, sem)

    def batch(t, _):
        @pl.when(t % 2 == 0)
        def _():
            process(t, 0, sem, sem2)

        @pl.when(t % 2 == 1)
        def _():
            process(t, 1, sem2, sem)
        return 0
    lax.fori_loop(0, NHB, batch, 0)
    pltpu.make_async_copy(lr_out.at[c, pl.ds(0, 2 * HB)],
                          gath_v.at[pl.ds(0, 2 * HB)], wsem).wait()


_sc_gather = functools.partial(
    pl.kernel,
    mesh=plsc.VectorSubcoreMesh(core_axis_name="c", subcore_axis_name="s"),
    compiler_params=pltpu.CompilerParams(needs_layout_passes=False),
    out_type=[
        jax.ShapeDtypeStruct((C, G_SEL, S_SEL), jnp.float32),
        jax.ShapeDtypeStruct((C, P, S_SEL), jnp.float32),
        jax.ShapeDtypeStruct((C, G_SEL, P), jnp.float32),
    ],
    scratch_types=[
        pltpu.VMEM((S_SEL,), jnp.int32),
        pltpu.VMEM((ROWS_PER_W,), jnp.int32),
        pltpu.VMEM((RB, S), jnp.float32),
        pltpu.VMEM((2 * RB, S_SEL), jnp.float32),
        pltpu.VMEM((VR, P), jnp.float32),
        pltpu.SemaphoreType.DMA,
        pltpu.SemaphoreType.DMA,
        pltpu.SemaphoreType.DMA,
    ],
)(_sc_body)


BG = 64


def _tc_body(lr_ref, w_ref, v_ref, o_ref):
    acc = jnp.zeros((BG, S_SEL), jnp.float32)
    for c in range(C):
        prog = jnp.dot(v_ref[c], w_ref[c], preferred_element_type=jnp.float32)
        acc = acc + jnp.exp(lr_ref[c] + prog)
    o_ref[...] = acc


_tc_combine = pl.pallas_call(
    _tc_body,
    grid=(G_SEL // BG,),
    in_specs=[
        pl.BlockSpec((C, BG, S_SEL), lambda i: (0, i, 0)),
        pl.BlockSpec((C, P, S_SEL), lambda i: (0, 0, 0)),
        pl.BlockSpec((C, BG, P), lambda i: (0, i, 0)),
    ],
    out_specs=pl.BlockSpec((BG, S_SEL), lambda i: (i, 0)),
    out_shape=jax.ShapeDtypeStruct((G_SEL, S_SEL), jnp.float32),
)


def kernel(log_rates, W, V, spots, genes):
    spots32 = spots.astype(jnp.int32)
    genes32 = genes.astype(jnp.int32)
    lr_sel, w_sel, v_sel = _sc_gather(log_rates, W, V, spots32, genes32)
    return _tc_combine(lr_sel, w_sel, v_sel)
